# baseline (speedup 1.0000x reference)
import functools

import jax
import jax.numpy as jnp
from jax import lax
from jax.experimental import pallas as pl
from jax.experimental.pallas import tpu as pltpu
from jax.experimental.pallas import tpu_sc as plsc

_N = 10000
_E = 320000
_D = 128
_H = 64
_TD = 80
_XD = 16

_NC, _NS = 2, 16
_NW = _NC * _NS
_P = 2
_ES = _E // _P
_EPW = _ES // _NW
_C = 40
_NCHUNK = _EPW // _C
_NBUF = 5
_BN = 1000
_BE = 2000


def _pre_body(h_ref, x_ref, vi_ref, we1a, be1, we1b, wn1a, wv1, bv1, wv2,
              t1_ref, t2_ref, hn_ref, vout_ref):
    hb = h_ref[...]
    xpad = jnp.concatenate(
        [x_ref[...], jnp.zeros((_BN, _XD - 3), jnp.float32)], axis=1)
    a1 = jnp.dot(hb, we1a[...], preferred_element_type=jnp.float32) + be1[...]
    a2 = jnp.dot(hb, we1b[...], preferred_element_type=jnp.float32)
    t1_ref[...] = jnp.concatenate([a1, xpad], axis=1)
    t2_ref[...] = jnp.concatenate([a2, xpad], axis=1)
    hn_ref[...] = jnp.dot(hb, wn1a[...], preferred_element_type=jnp.float32)
    vs = jax.nn.silu(
        jnp.dot(hb, wv1[...], preferred_element_type=jnp.float32) + bv1[...])
    vel = jnp.sum(vs * wv2[...], axis=1, keepdims=True)
    vout_ref[...] = vi_ref[...] * vel


def _tc_pre(h, x, v_init, we1a, be1, we1b, wn1a, wv1, bv1, wv2):
    grid = (_N // _BN,)
    full = lambda r, c: pl.BlockSpec((r, c), lambda i: (0, 0))
    return pl.pallas_call(
        _pre_body,
        grid=grid,
        in_specs=[
            pl.BlockSpec((_BN, _D), lambda i: (i, 0)),
            pl.BlockSpec((_BN, 3), lambda i: (i, 0)),
            pl.BlockSpec((_BN, 3), lambda i: (i, 0)),
            full(_D, _H), full(1, _H), full(_D, _H), full(_D, _H),
            full(_D, _H), full(1, _H), full(1, _H),
        ],
        out_specs=[
            pl.BlockSpec((_BN, _TD), lambda i: (i, 0)),
            pl.BlockSpec((_BN, _TD), lambda i: (i, 0)),
            pl.BlockSpec((_BN, _H), lambda i: (i, 0)),
            pl.BlockSpec((_BN, 3), lambda i: (i, 0)),
        ],
        out_shape=[
            jax.ShapeDtypeStruct((_N, _TD), jnp.float32),
            jax.ShapeDtypeStruct((_N, _TD), jnp.float32),
            jax.ShapeDtypeStruct((_N, _H), jnp.float32),
            jax.ShapeDtypeStruct((_N, 3), jnp.float32),
        ],
    )(h, x, v_init, we1a, be1, we1b, wn1a, wv1, bv1, wv2)


_GBUF = 5


def _sc_gather(row, col, ea, t1, t2, ebase):
    mesh = plsc.VectorSubcoreMesh(core_axis_name="c", subcore_axis_name="s")
    scratch = (
        [pltpu.VMEM((_EPW,), jnp.int32) for _ in range(2)]
        + [pltpu.VMEM((_C, _TD), jnp.float32) for _ in range(2 * _GBUF)]
        + [pltpu.VMEM((_C, _D), jnp.float32) for _ in range(_GBUF)]
        + [pltpu.SemaphoreType.DMA for _ in range(3 * _GBUF)]
    )

    @functools.partial(
        pl.kernel,
        out_type=jax.ShapeDtypeStruct((_ES, _D), jnp.float32),
        mesh=mesh,
        scratch_types=scratch,
        compiler_params=pltpu.CompilerParams(use_tc_tiling_on_sc=False),
    )
    def k(row_h, col_h, ea_h, t1_h, t2_h, mx_h, *s):
        idx_r, idx_c = s[0], s[1]
        bufs1 = s[2:2 + _GBUF]
        bufs2 = s[2 + _GBUF:2 + 2 * _GBUF]
        outb = s[2 + 2 * _GBUF:2 + 3 * _GBUF]
        gsem = s[2 + 3 * _GBUF:2 + 4 * _GBUF]
        wsem = s[2 + 4 * _GBUF:2 + 5 * _GBUF]
        esem = s[2 + 5 * _GBUF:2 + 6 * _GBUF]
        cid = lax.axis_index("c")
        sid = lax.axis_index("s")
        wid = cid * _NS + sid
        base = wid * _EPW
        pltpu.sync_copy(row_h.at[pl.ds(ebase + base, _EPW)], idx_r)
        pltpu.sync_copy(col_h.at[pl.ds(ebase + base, _EPW)], idx_c)

        def issue_get(j, b):
            ofs = j * _C
            pltpu.async_copy(
                t1_h.at[idx_r.at[pl.ds(ofs, _C)]], bufs1[b], gsem[b])
            pltpu.async_copy(
                t2_h.at[idx_c.at[pl.ds(ofs, _C)]], bufs2[b], gsem[b])

        def wait_get(b):
            pltpu.make_async_copy(t1_h.at[pl.ds(0, _C)], bufs1[b],
                                  gsem[b]).wait()
            pltpu.make_async_copy(t2_h.at[pl.ds(0, _C)], bufs2[b],
                                  gsem[b]).wait()

        def wait_put(b):
            pltpu.make_async_copy(outb[b], mx_h.at[pl.ds(0, _C)],
                                  wsem[b]).wait()

        def compute(j, b):
            ecol = pltpu.async_copy(
                ea_h.at[pl.ds(ebase + base + j * _C, _C)],
                outb[b].at[pl.ds(0, _C), pl.ds(_TD, 1)], esem[b])

            def edge(i, carry):
                d16 = (bufs1[b][i, pl.ds(_H, 16)]
                       - bufs2[b][i, pl.ds(_H, 16)])
                for q in range(4):
                    sl = pl.ds(q * 16, 16)
                    outb[b][i, sl] = bufs1[b][i, sl] + bufs2[b][i, sl]
                outb[b][i, pl.ds(_H, 16)] = d16
                return carry

            lax.fori_loop(0, _C, edge, 0)
            ecol.wait()
            pltpu.async_copy(outb[b], mx_h.at[pl.ds(base + j * _C, _C)],
                             wsem[b])

        for b in range(_GBUF):
            issue_get(b, b)

        def group(g, carry):
            for b in range(_GBUF):
                j = g * _GBUF + b
                wait_get(b)

                @pl.when(g > 0)
                def _drain():
                    wait_put(b)

                compute(j, b)

                @pl.when(j + _GBUF < _NCHUNK)
                def _prefetch():
                    issue_get(j + _GBUF, b)

            return carry

        lax.fori_loop(0, _NCHUNK // _GBUF, group, 0)
        for b in range(_GBUF):
            wait_put(b)

    return k(row, col, ea, t1, t2)


def _sigm(x):
    return 0.5 * jnp.tanh(0.5 * x) + 0.5


def _silu(x):
    return x * _sigm(x)


def _edge_body(g_ref, k17, sel3, we2, be2, wi1, bi1, wi2, bi2s,
               wc1, bc1, wc2, mx_ref):
    g = g_ref[...]
    ds = g[:, _H:_H + 16]
    dsea = jnp.concatenate([ds * ds, g[:, _TD:_TD + 1]], axis=1)
    epre = (g[:, :_H]
            + jnp.dot(dsea, k17[...], preferred_element_type=jnp.float32))
    m1 = _silu(epre)
    m = _silu(
        jnp.dot(m1, we2[...], preferred_element_type=jnp.float32) + be2[...])
    i1 = _silu(
        jnp.dot(m, wi1[...], preferred_element_type=jnp.float32) + bi1[...])
    e = _sigm(jnp.dot(i1, wi2[...], preferred_element_type=jnp.float32)
              + bi2s[...])
    c1 = _silu(
        jnp.dot(m, wc1[...], preferred_element_type=jnp.float32) + bc1[...])
    phi = jnp.dot(c1, wc2[...], preferred_element_type=jnp.float32)
    xu16 = (e * phi) * ds + sel3[...]
    mx_ref[...] = jnp.concatenate(
        [e * m, xu16, jnp.zeros((_BE, _D - _H - 16), jnp.float32)], axis=1)


def _tc_edge(g, k17, sel3, we2, be2, wi1, bi1, wi2, bi2s, wc1, bc1, wc2):
    grid = (_ES // _BE,)
    full = lambda r, c: pl.BlockSpec((r, c), lambda i: (0, 0))
    return pl.pallas_call(
        _edge_body,
        grid=grid,
        in_specs=[
            pl.BlockSpec((_BE, _D), lambda i: (i, 0)),
            full(17, _H), full(1, 16), full(_H, _H), full(1, _H),
            full(_H, _H // 2), full(1, _H // 2), full(_H // 2, 1), full(1, 1),
            full(_H, _H), full(1, _H), full(_H, 1),
        ],
        out_specs=pl.BlockSpec((_BE, _D), lambda i: (i, 0)),
        out_shape=jax.ShapeDtypeStruct((_ES, _D), jnp.float32),
    )(g, k17, sel3, we2, be2, wi1, bi1, wi2, bi2s, wc1, bc1, wc2)


def _sc_scatter(row, mx, zm, ebase):
    mesh = plsc.VectorSubcoreMesh(core_axis_name="c", subcore_axis_name="s")
    scratch = (
        [pltpu.VMEM((_C,), jnp.int32) for _ in range(_NBUF)]
        + [pltpu.VMEM((_C, _TD), jnp.float32) for _ in range(_NBUF)]
        + [pltpu.SemaphoreType.DMA for _ in range(_NBUF)]
        + [pltpu.VMEM_SHARED((_N, _TD), jnp.float32)]
    )

    @functools.partial(
        pl.kernel,
        out_type=jax.ShapeDtypeStruct((_NC, _N, _D), jnp.float32),
        mesh=mesh,
        scratch_types=scratch,
        compiler_params=pltpu.CompilerParams(use_tc_tiling_on_sc=False),
    )
    def k(row_h, mx_h, zm_h, macc_h, *s):
        ibuf = s[0:_NBUF]
        mbuf = s[_NBUF:2 * _NBUF]
        sem = s[2 * _NBUF:3 * _NBUF]
        sh_m = s[3 * _NBUF]
        cid = lax.axis_index("c")
        sid = lax.axis_index("s")
        wid = cid * _NS + sid
        base = wid * _EPW

        @pl.when(sid == 0)
        def _init():
            pltpu.sync_copy(zm_h, sh_m)

        plsc.subcore_barrier()

        def group(g, carry):
            gets = []
            for b in range(_NBUF):
                j = g * _NBUF + b
                src = base + j * _C
                gets.append(pltpu.async_copy(
                    row_h.at[pl.ds(ebase + src, _C)], ibuf[b], sem[b]))
                gets.append(pltpu.async_copy(
                    mx_h.at[pl.ds(src, _C), pl.ds(0, _TD)], mbuf[b], sem[b]))
            for b in range(_NBUF):
                gets[2 * b].wait()
                gets[2 * b + 1].wait()
                pltpu.sync_copy(mbuf[b], sh_m.at[ibuf[b]], add=True)
            return carry

        lax.fori_loop(0, _NCHUNK // _NBUF, group, 0)
        plsc.subcore_barrier()

        @pl.when(sid == 0)
        def _writeout():
            pltpu.sync_copy(sh_m, macc_h.at[cid, :, pl.ds(0, _TD)])

    return k(row, mx, zm)


def _final_body(hn_ref, *refs):
    (macc_refs, (x_ref, vout_ref, wn1b, bn1, wn2, bn2,
                 hout_ref, xout_ref)) = refs[:_P], refs[_P:]
    acc = macc_refs[0][0] + macc_refs[0][1]
    for r in macc_refs[1:]:
        acc = acc + r[0] + r[1]
    m_i = acc[:, :_H]
    t = jax.nn.silu(
        hn_ref[...]
        + jnp.dot(m_i, wn1b[...], preferred_element_type=jnp.float32)
        + bn1[...])
    hout_ref[...] = jnp.dot(t, wn2[...],
                            preferred_element_type=jnp.float32) + bn2[...]
    deg = acc[:, _H + 3:_H + 4]
    aggx = acc[:, _H:_H + 3]
    xb = x_ref[...]
    xout_ref[...] = jnp.where(
        deg > 0.0, xb + vout_ref[...] + aggx / jnp.float32(_N - 1), xb)


def _tc_final(hn, maccs, x, v_out, wn1b, bn1, wn2, bn2):
    grid = (_N // _BN,)
    full = lambda r, c: pl.BlockSpec((r, c), lambda i: (0, 0))
    return pl.pallas_call(
        _final_body,
        grid=grid,
        in_specs=[
            pl.BlockSpec((_BN, _H), lambda i: (i, 0)),
        ] + [
            pl.BlockSpec((_NC, _BN, _D), lambda i: (0, i, 0))
            for _ in range(_P)
        ] + [
            pl.BlockSpec((_BN, 3), lambda i: (i, 0)),
            pl.BlockSpec((_BN, 3), lambda i: (i, 0)),
            full(_H, _H), full(1, _H), full(_H, _D), full(1, _D),
        ],
        out_specs=[
            pl.BlockSpec((_BN, _D), lambda i: (i, 0)),
            pl.BlockSpec((_BN, 3), lambda i: (i, 0)),
        ],
        out_shape=[
            jax.ShapeDtypeStruct((_N, _D), jnp.float32),
            jax.ShapeDtypeStruct((_N, 3), jnp.float32),
        ],
    )(hn, *maccs, x, v_out, wn1b, bn1, wn2, bn2)


def kernel(h, x, edge_index, edge_attr, v_init, We1, be1, We2, be2, Wc1, bc1,
           Wc2, Wn1, bn1, Wn2, bn2, Wv1, bv1, Wv2, Wi1, bi1, Wi2, bi2):
    f32 = jnp.float32
    row = edge_index[0]
    col = edge_index[1]

    t1, t2, hn, v_out = _tc_pre(
        h, x, v_init,
        We1[:_D], be1.reshape(1, _H), We1[_D:2 * _D], Wn1[:_D],
        Wv1, bv1.reshape(1, _H), Wv2[:, 0].reshape(1, _H))

    k17 = jnp.concatenate(
        [jnp.tile(We1[2 * _D].reshape(1, _H), (3, 1)),
         jnp.zeros((13, _H), f32),
         We1[2 * _D + 1].reshape(1, _H)], axis=0)
    sel3 = jnp.zeros((1, 16), f32).at[0, 3].set(1.0)
    zm = jnp.zeros((_N, _TD), f32)

    maccs = []
    for s in range(_P):
        gx = _sc_gather(row, col, edge_attr, t1, t2, s * _ES)
        mx = _tc_edge(
            gx,
            k17, sel3,
            We2, be2.reshape(1, _H),
            Wi1, bi1.reshape(1, _H // 2), Wi2,
            bi2.reshape(1, 1),
            Wc1, bc1.reshape(1, _H), Wc2)
        maccs.append(_sc_scatter(row, mx, zm, s * _ES))

    h_out, x_out = _tc_final(
        hn, maccs, x, v_out,
        Wn1[_D:], bn1.reshape(1, _H), Wn2, bn2.reshape(1, _D))

    return (h_out, x_out, v_out)


# speedup vs baseline: 1.4810x; 1.4810x over previous
import functools

import jax
import jax.numpy as jnp
from jax import lax
from jax.experimental import pallas as pl
from jax.experimental.pallas import tpu as pltpu
from jax.experimental.pallas import tpu_sc as plsc

_N = 10000
_E = 320000
_D = 128
_H = 64
_TD = 80
_XD = 16

_NC, _NS = 2, 16
_NW = _NC * _NS
_P = 2
_ES = _E // _P
_EPW = _ES // _NW
_C = 40
_NCHUNK = _EPW // _C
_NBUF = 5
_BN = 1000
_BE = 2000


def _pre_body(h_ref, x_ref, vi_ref, we1a, be1, we1b, wn1a, wv1, bv1, wv2,
              t1_ref, t2_ref, hn_ref, vout_ref):
    hb = h_ref[...]
    xpad = jnp.concatenate(
        [x_ref[...], jnp.zeros((_BN, _XD - 3), jnp.float32)], axis=1)
    a1 = jnp.dot(hb, we1a[...], preferred_element_type=jnp.float32) + be1[...]
    a2 = jnp.dot(hb, we1b[...], preferred_element_type=jnp.float32)
    t1_ref[...] = jnp.concatenate([a1, xpad], axis=1)
    t2_ref[...] = jnp.concatenate([a2, xpad], axis=1)
    hn_ref[...] = jnp.dot(hb, wn1a[...], preferred_element_type=jnp.float32)
    vs = jax.nn.silu(
        jnp.dot(hb, wv1[...], preferred_element_type=jnp.float32) + bv1[...])
    vel = jnp.sum(vs * wv2[...], axis=1, keepdims=True)
    vout_ref[...] = vi_ref[...] * vel


def _tc_pre(h, x, v_init, we1a, be1, we1b, wn1a, wv1, bv1, wv2):
    grid = (_N // _BN,)
    full = lambda r, c: pl.BlockSpec((r, c), lambda i: (0, 0))
    return pl.pallas_call(
        _pre_body,
        grid=grid,
        in_specs=[
            pl.BlockSpec((_BN, _D), lambda i: (i, 0)),
            pl.BlockSpec((_BN, 3), lambda i: (i, 0)),
            pl.BlockSpec((_BN, 3), lambda i: (i, 0)),
            full(_D, _H), full(1, _H), full(_D, _H), full(_D, _H),
            full(_D, _H), full(1, _H), full(1, _H),
        ],
        out_specs=[
            pl.BlockSpec((_BN, _TD), lambda i: (i, 0)),
            pl.BlockSpec((_BN, _TD), lambda i: (i, 0)),
            pl.BlockSpec((_BN, _H), lambda i: (i, 0)),
            pl.BlockSpec((_BN, 3), lambda i: (i, 0)),
        ],
        out_shape=[
            jax.ShapeDtypeStruct((_N, _TD), jnp.float32),
            jax.ShapeDtypeStruct((_N, _TD), jnp.float32),
            jax.ShapeDtypeStruct((_N, _H), jnp.float32),
            jax.ShapeDtypeStruct((_N, 3), jnp.float32),
        ],
    )(h, x, v_init, we1a, be1, we1b, wn1a, wv1, bv1, wv2)


_GBUF = 5


def _sc_gather(row, col, ea, t1, t2, ebase):
    mesh = plsc.VectorSubcoreMesh(core_axis_name="c", subcore_axis_name="s")
    scratch = (
        [pltpu.VMEM((_EPW,), jnp.int32) for _ in range(2)]
        + [pltpu.VMEM((_C, _TD), jnp.float32) for _ in range(2 * _GBUF)]
        + [pltpu.VMEM((_C, _D), jnp.float32) for _ in range(_GBUF)]
        + [pltpu.VMEM((_C + 8,), jnp.float32) for _ in range(_GBUF)]
        + [pltpu.SemaphoreType.DMA for _ in range(2 * _GBUF)]
    )

    @functools.partial(
        pl.kernel,
        out_type=jax.ShapeDtypeStruct((_ES, _D), jnp.float32),
        mesh=mesh,
        scratch_types=scratch,
        compiler_params=pltpu.CompilerParams(use_tc_tiling_on_sc=False,
                                             needs_layout_passes=False),
    )
    def k(row_h, col_h, ea_h, t1_h, t2_h, mx_h, *s):
        idx_r, idx_c = s[0], s[1]
        bufs1 = s[2:2 + _GBUF]
        bufs2 = s[2 + _GBUF:2 + 2 * _GBUF]
        outb = s[2 + 2 * _GBUF:2 + 3 * _GBUF]
        eab = s[2 + 3 * _GBUF:2 + 4 * _GBUF]
        gsem = s[2 + 4 * _GBUF:2 + 5 * _GBUF]
        wsem = s[2 + 5 * _GBUF:2 + 6 * _GBUF]
        cid = lax.axis_index("c")
        sid = lax.axis_index("s")
        wid = cid * _NS + sid
        base = wid * _EPW
        pltpu.sync_copy(row_h.at[pl.ds(ebase + base, _EPW)], idx_r)
        pltpu.sync_copy(col_h.at[pl.ds(ebase + base, _EPW)], idx_c)
        iota16 = lax.iota(jnp.int32, 16)
        lane80 = jnp.full((16,), _TD, jnp.int32)
        m8 = iota16 < 8

        def issue_get(j, b):
            ofs = j * _C
            pltpu.async_copy(
                t1_h.at[idx_r.at[pl.ds(ofs, _C)]], bufs1[b], gsem[b])
            pltpu.async_copy(
                t2_h.at[idx_c.at[pl.ds(ofs, _C)]], bufs2[b], gsem[b])
            pltpu.async_copy(
                ea_h.at[pl.ds(ebase + base + ofs, _C)],
                eab[b].at[pl.ds(0, _C)], gsem[b])

        def wait_get(b):
            pltpu.make_async_copy(t1_h.at[pl.ds(0, _C)], bufs1[b],
                                  gsem[b]).wait()
            pltpu.make_async_copy(t2_h.at[pl.ds(0, _C)], bufs2[b],
                                  gsem[b]).wait()
            pltpu.make_async_copy(ea_h.at[pl.ds(0, _C)],
                                  eab[b].at[pl.ds(0, _C)], gsem[b]).wait()

        def wait_put(b):
            pltpu.make_async_copy(outb[b], mx_h.at[pl.ds(0, _C)],
                                  wsem[b]).wait()

        def compute(j, b):
            def edge(i, carry):
                d16 = (bufs1[b][i, pl.ds(_H, 16)]
                       - bufs2[b][i, pl.ds(_H, 16)])
                for q in range(4):
                    sl = pl.ds(q * 16, 16)
                    outb[b][i, sl] = bufs1[b][i, sl] + bufs2[b][i, sl]
                outb[b][i, pl.ds(_H, 16)] = d16
                return carry

            lax.fori_loop(0, _C, edge, 0)

            def eascat(k8, carry):
                ea16 = eab[b][pl.ds(k8 * 8, 16)]
                plsc.store_scatter(outb[b], [iota16 + k8 * 8, lane80],
                                   ea16, mask=m8)
                return carry

            lax.fori_loop(0, _C // 8, eascat, 0)
            pltpu.async_copy(outb[b], mx_h.at[pl.ds(base + j * _C, _C)],
                             wsem[b])

        for b in range(_GBUF):
            issue_get(b, b)

        def group(g, carry):
            for b in range(_GBUF):
                j = g * _GBUF + b
                wait_get(b)

                @pl.when(g > 0)
                def _drain():
                    wait_put(b)

                compute(j, b)

                @pl.when(j + _GBUF < _NCHUNK)
                def _prefetch():
                    issue_get(j + _GBUF, b)

            return carry

        lax.fori_loop(0, _NCHUNK // _GBUF, group, 0)
        for b in range(_GBUF):
            wait_put(b)

    return k(row, col, ea, t1, t2)


def _sigm(x):
    return 0.5 * jnp.tanh(0.5 * x) + 0.5


def _silu(x):
    return x * _sigm(x)


def _edge_body(g_ref, k17, sel3, we2, be2, wi1, bi1, wi2, bi2s,
               wc1, bc1, wc2, mx_ref):
    g = g_ref[...]
    ds = g[:, _H:_H + 16]
    dsea = jnp.concatenate([ds * ds, g[:, _TD:_TD + 1]], axis=1)
    epre = (g[:, :_H]
            + jnp.dot(dsea, k17[...], preferred_element_type=jnp.float32))
    m1 = _silu(epre)
    m = _silu(
        jnp.dot(m1, we2[...], preferred_element_type=jnp.float32) + be2[...])
    i1 = _silu(
        jnp.dot(m, wi1[...], preferred_element_type=jnp.float32) + bi1[...])
    e = _sigm(jnp.dot(i1, wi2[...], preferred_element_type=jnp.float32)
              + bi2s[...])
    c1 = _silu(
        jnp.dot(m, wc1[...], preferred_element_type=jnp.float32) + bc1[...])
    phi = jnp.dot(c1, wc2[...], preferred_element_type=jnp.float32)
    xu16 = (e * phi) * ds + sel3[...]
    mx_ref[...] = jnp.concatenate(
        [e * m, xu16, jnp.zeros((_BE, _D - _H - 16), jnp.float32)], axis=1)


def _tc_edge(g, k17, sel3, we2, be2, wi1, bi1, wi2, bi2s, wc1, bc1, wc2):
    grid = (_ES // _BE,)
    full = lambda r, c: pl.BlockSpec((r, c), lambda i: (0, 0))
    return pl.pallas_call(
        _edge_body,
        grid=grid,
        in_specs=[
            pl.BlockSpec((_BE, _D), lambda i: (i, 0)),
            full(17, _H), full(1, 16), full(_H, _H), full(1, _H),
            full(_H, _H // 2), full(1, _H // 2), full(_H // 2, 1), full(1, 1),
            full(_H, _H), full(1, _H), full(_H, 1),
        ],
        out_specs=pl.BlockSpec((_BE, _D), lambda i: (i, 0)),
        out_shape=jax.ShapeDtypeStruct((_ES, _D), jnp.float32),
    )(g, k17, sel3, we2, be2, wi1, bi1, wi2, bi2s, wc1, bc1, wc2)


def _sc_scatter(row, mx, zm, ebase):
    mesh = plsc.VectorSubcoreMesh(core_axis_name="c", subcore_axis_name="s")
    scratch = (
        [pltpu.VMEM((_C,), jnp.int32) for _ in range(_NBUF)]
        + [pltpu.VMEM((_C, _TD), jnp.float32) for _ in range(_NBUF)]
        + [pltpu.SemaphoreType.DMA for _ in range(_NBUF)]
        + [pltpu.VMEM_SHARED((_N, _TD), jnp.float32)]
    )

    @functools.partial(
        pl.kernel,
        out_type=jax.ShapeDtypeStruct((_NC, _N, _D), jnp.float32),
        mesh=mesh,
        scratch_types=scratch,
        compiler_params=pltpu.CompilerParams(use_tc_tiling_on_sc=False),
    )
    def k(row_h, mx_h, zm_h, macc_h, *s):
        ibuf = s[0:_NBUF]
        mbuf = s[_NBUF:2 * _NBUF]
        sem = s[2 * _NBUF:3 * _NBUF]
        sh_m = s[3 * _NBUF]
        cid = lax.axis_index("c")
        sid = lax.axis_index("s")
        wid = cid * _NS + sid
        base = wid * _EPW

        @pl.when(sid == 0)
        def _init():
            pltpu.sync_copy(zm_h, sh_m)

        plsc.subcore_barrier()

        def group(g, carry):
            gets = []
            for b in range(_NBUF):
                j = g * _NBUF + b
                src = base + j * _C
                gets.append(pltpu.async_copy(
                    row_h.at[pl.ds(ebase + src, _C)], ibuf[b], sem[b]))
                gets.append(pltpu.async_copy(
                    mx_h.at[pl.ds(src, _C), pl.ds(0, _TD)], mbuf[b], sem[b]))
            for b in range(_NBUF):
                gets[2 * b].wait()
                gets[2 * b + 1].wait()
                pltpu.sync_copy(mbuf[b], sh_m.at[ibuf[b]], add=True)
            return carry

        lax.fori_loop(0, _NCHUNK // _NBUF, group, 0)
        plsc.subcore_barrier()

        @pl.when(sid == 0)
        def _writeout():
            pltpu.sync_copy(sh_m, macc_h.at[cid, :, pl.ds(0, _TD)])

    return k(row, mx, zm)


def _final_body(hn_ref, *refs):
    (macc_refs, (x_ref, vout_ref, wn1b, bn1, wn2, bn2,
                 hout_ref, xout_ref)) = refs[:_P], refs[_P:]
    acc = macc_refs[0][0] + macc_refs[0][1]
    for r in macc_refs[1:]:
        acc = acc + r[0] + r[1]
    m_i = acc[:, :_H]
    t = jax.nn.silu(
        hn_ref[...]
        + jnp.dot(m_i, wn1b[...], preferred_element_type=jnp.float32)
        + bn1[...])
    hout_ref[...] = jnp.dot(t, wn2[...],
                            preferred_element_type=jnp.float32) + bn2[...]
    deg = acc[:, _H + 3:_H + 4]
    aggx = acc[:, _H:_H + 3]
    xb = x_ref[...]
    xout_ref[...] = jnp.where(
        deg > 0.0, xb + vout_ref[...] + aggx / jnp.float32(_N - 1), xb)


def _tc_final(hn, maccs, x, v_out, wn1b, bn1, wn2, bn2):
    grid = (_N // _BN,)
    full = lambda r, c: pl.BlockSpec((r, c), lambda i: (0, 0))
    return pl.pallas_call(
        _final_body,
        grid=grid,
        in_specs=[
            pl.BlockSpec((_BN, _H), lambda i: (i, 0)),
        ] + [
            pl.BlockSpec((_NC, _BN, _D), lambda i: (0, i, 0))
            for _ in range(_P)
        ] + [
            pl.BlockSpec((_BN, 3), lambda i: (i, 0)),
            pl.BlockSpec((_BN, 3), lambda i: (i, 0)),
            full(_H, _H), full(1, _H), full(_H, _D), full(1, _D),
        ],
        out_specs=[
            pl.BlockSpec((_BN, _D), lambda i: (i, 0)),
            pl.BlockSpec((_BN, 3), lambda i: (i, 0)),
        ],
        out_shape=[
            jax.ShapeDtypeStruct((_N, _D), jnp.float32),
            jax.ShapeDtypeStruct((_N, 3), jnp.float32),
        ],
    )(hn, *maccs, x, v_out, wn1b, bn1, wn2, bn2)


def kernel(h, x, edge_index, edge_attr, v_init, We1, be1, We2, be2, Wc1, bc1,
           Wc2, Wn1, bn1, Wn2, bn2, Wv1, bv1, Wv2, Wi1, bi1, Wi2, bi2):
    f32 = jnp.float32
    row = edge_index[0]
    col = edge_index[1]

    t1, t2, hn, v_out = _tc_pre(
        h, x, v_init,
        We1[:_D], be1.reshape(1, _H), We1[_D:2 * _D], Wn1[:_D],
        Wv1, bv1.reshape(1, _H), Wv2[:, 0].reshape(1, _H))

    k17 = jnp.concatenate(
        [jnp.tile(We1[2 * _D].reshape(1, _H), (3, 1)),
         jnp.zeros((13, _H), f32),
         We1[2 * _D + 1].reshape(1, _H)], axis=0)
    sel3 = jnp.zeros((1, 16), f32).at[0, 3].set(1.0)
    zm = jnp.zeros((_N, _TD), f32)

    maccs = []
    for s in range(_P):
        gx = _sc_gather(row, col, edge_attr.reshape(_E), t1, t2, s * _ES)
        mx = _tc_edge(
            gx,
            k17, sel3,
            We2, be2.reshape(1, _H),
            Wi1, bi1.reshape(1, _H // 2), Wi2,
            bi2.reshape(1, 1),
            Wc1, bc1.reshape(1, _H), Wc2)
        maccs.append(_sc_scatter(row, mx, zm, s * _ES))

    h_out, x_out = _tc_final(
        hn, maccs, x, v_out,
        Wn1[_D:], bn1.reshape(1, _H), Wn2, bn2.reshape(1, _D))

    return (h_out, x_out, v_out)


# speedup vs baseline: 1.5656x; 1.0571x over previous
import functools

import jax
import jax.numpy as jnp
from jax import lax
from jax.experimental import pallas as pl
from jax.experimental.pallas import tpu as pltpu
from jax.experimental.pallas import tpu_sc as plsc

_N = 10000
_E = 320000
_D = 128
_H = 64
_TD = 80
_XD = 16

_NC, _NS = 2, 16
_NW = _NC * _NS
_SLICES = (96000, 128000, 96000)
_P = len(_SLICES)
_C = 40
_NBUF = 5
_BN = 1000
_BE = 2000


def _pre_body(h_ref, x_ref, vi_ref, we1a, be1, we1b, wn1a, wv1, bv1, wv2,
              t1_ref, t2_ref, hn_ref, vout_ref):
    hb = h_ref[...]
    xpad = jnp.concatenate(
        [x_ref[...], jnp.zeros((_BN, _XD - 3), jnp.float32)], axis=1)
    a1 = jnp.dot(hb, we1a[...], preferred_element_type=jnp.float32) + be1[...]
    a2 = jnp.dot(hb, we1b[...], preferred_element_type=jnp.float32)
    t1_ref[...] = jnp.concatenate([a1, xpad], axis=1)
    t2_ref[...] = jnp.concatenate([a2, xpad], axis=1)
    hn_ref[...] = jnp.dot(hb, wn1a[...], preferred_element_type=jnp.float32)
    vs = jax.nn.silu(
        jnp.dot(hb, wv1[...], preferred_element_type=jnp.float32) + bv1[...])
    vel = jnp.sum(vs * wv2[...], axis=1, keepdims=True)
    vout_ref[...] = vi_ref[...] * vel


def _tc_pre(h, x, v_init, we1a, be1, we1b, wn1a, wv1, bv1, wv2):
    grid = (_N // _BN,)
    full = lambda r, c: pl.BlockSpec((r, c), lambda i: (0, 0))
    return pl.pallas_call(
        _pre_body,
        grid=grid,
        in_specs=[
            pl.BlockSpec((_BN, _D), lambda i: (i, 0)),
            pl.BlockSpec((_BN, 3), lambda i: (i, 0)),
            pl.BlockSpec((_BN, 3), lambda i: (i, 0)),
            full(_D, _H), full(1, _H), full(_D, _H), full(_D, _H),
            full(_D, _H), full(1, _H), full(1, _H),
        ],
        out_specs=[
            pl.BlockSpec((_BN, _TD), lambda i: (i, 0)),
            pl.BlockSpec((_BN, _TD), lambda i: (i, 0)),
            pl.BlockSpec((_BN, _H), lambda i: (i, 0)),
            pl.BlockSpec((_BN, 3), lambda i: (i, 0)),
        ],
        out_shape=[
            jax.ShapeDtypeStruct((_N, _TD), jnp.float32),
            jax.ShapeDtypeStruct((_N, _TD), jnp.float32),
            jax.ShapeDtypeStruct((_N, _H), jnp.float32),
            jax.ShapeDtypeStruct((_N, 3), jnp.float32),
        ],
    )(h, x, v_init, we1a, be1, we1b, wn1a, wv1, bv1, wv2)


_GBUF = 5


def _sc_gather(row, col, ea, t1, t2, ebase, es):
    epw = es // _NW
    nchunk = epw // _C
    mesh = plsc.VectorSubcoreMesh(core_axis_name="c", subcore_axis_name="s")
    scratch = (
        [pltpu.VMEM((epw,), jnp.int32) for _ in range(2)]
        + [pltpu.VMEM((_C, _TD), jnp.float32) for _ in range(2 * _GBUF)]
        + [pltpu.VMEM((_C, _D), jnp.float32) for _ in range(_GBUF)]
        + [pltpu.VMEM((_C + 8,), jnp.float32) for _ in range(_GBUF)]
        + [pltpu.SemaphoreType.DMA for _ in range(2 * _GBUF)]
    )

    @functools.partial(
        pl.kernel,
        out_type=jax.ShapeDtypeStruct((es, _D), jnp.float32),
        mesh=mesh,
        scratch_types=scratch,
        compiler_params=pltpu.CompilerParams(use_tc_tiling_on_sc=False,
                                             needs_layout_passes=False),
    )
    def k(row_h, col_h, ea_h, t1_h, t2_h, mx_h, *s):
        idx_r, idx_c = s[0], s[1]
        bufs1 = s[2:2 + _GBUF]
        bufs2 = s[2 + _GBUF:2 + 2 * _GBUF]
        outb = s[2 + 2 * _GBUF:2 + 3 * _GBUF]
        eab = s[2 + 3 * _GBUF:2 + 4 * _GBUF]
        gsem = s[2 + 4 * _GBUF:2 + 5 * _GBUF]
        wsem = s[2 + 5 * _GBUF:2 + 6 * _GBUF]
        cid = lax.axis_index("c")
        sid = lax.axis_index("s")
        wid = cid * _NS + sid
        base = wid * epw
        pltpu.sync_copy(row_h.at[pl.ds(ebase + base, epw)], idx_r)
        pltpu.sync_copy(col_h.at[pl.ds(ebase + base, epw)], idx_c)
        iota16 = lax.iota(jnp.int32, 16)
        lane80 = jnp.full((16,), _TD, jnp.int32)
        m8 = iota16 < 8

        def issue_get(j, b):
            ofs = j * _C
            pltpu.async_copy(
                t1_h.at[idx_r.at[pl.ds(ofs, _C)]], bufs1[b], gsem[b])
            pltpu.async_copy(
                t2_h.at[idx_c.at[pl.ds(ofs, _C)]], bufs2[b], gsem[b])
            pltpu.async_copy(
                ea_h.at[pl.ds(ebase + base + ofs, _C)],
                eab[b].at[pl.ds(0, _C)], gsem[b])

        def wait_get(b):
            pltpu.make_async_copy(t1_h.at[pl.ds(0, _C)], bufs1[b],
                                  gsem[b]).wait()
            pltpu.make_async_copy(t2_h.at[pl.ds(0, _C)], bufs2[b],
                                  gsem[b]).wait()
            pltpu.make_async_copy(ea_h.at[pl.ds(0, _C)],
                                  eab[b].at[pl.ds(0, _C)], gsem[b]).wait()

        def wait_put(b):
            pltpu.make_async_copy(outb[b], mx_h.at[pl.ds(0, _C)],
                                  wsem[b]).wait()

        def compute(j, b):
            def edge(i, carry):
                d16 = (bufs1[b][i, pl.ds(_H, 16)]
                       - bufs2[b][i, pl.ds(_H, 16)])
                for q in range(4):
                    sl = pl.ds(q * 16, 16)
                    outb[b][i, sl] = bufs1[b][i, sl] + bufs2[b][i, sl]
                outb[b][i, pl.ds(_H, 16)] = d16
                return carry

            lax.fori_loop(0, _C, edge, 0)

            def eascat(k8, carry):
                ea16 = eab[b][pl.ds(k8 * 8, 16)]
                plsc.store_scatter(outb[b], [iota16 + k8 * 8, lane80],
                                   ea16, mask=m8)
                return carry

            lax.fori_loop(0, _C // 8, eascat, 0)
            pltpu.async_copy(outb[b], mx_h.at[pl.ds(base + j * _C, _C)],
                             wsem[b])

        for b in range(_GBUF):
            issue_get(b, b)

        def group(g, carry):
            for b in range(_GBUF):
                j = g * _GBUF + b
                wait_get(b)

                @pl.when(g > 0)
                def _drain():
                    wait_put(b)

                compute(j, b)

                @pl.when(j + _GBUF < nchunk)
                def _prefetch():
                    issue_get(j + _GBUF, b)

            return carry

        lax.fori_loop(0, nchunk // _GBUF, group, 0)
        for b in range(_GBUF):
            wait_put(b)

    return k(row, col, ea, t1, t2)


def _sigm(x):
    return 0.5 * jnp.tanh(0.5 * x) + 0.5


def _silu(x):
    return x * _sigm(x)


def _edge_body(g_ref, k17, sel3, we2, be2, wi1, bi1, wi2, bi2s,
               wc1, bc1, wc2, mx_ref):
    g = g_ref[...]
    ds = g[:, _H:_H + 16]
    dsea = jnp.concatenate([ds * ds, g[:, _TD:_TD + 1]], axis=1)
    epre = (g[:, :_H]
            + jnp.dot(dsea, k17[...], preferred_element_type=jnp.float32))
    m1 = _silu(epre)
    m = _silu(
        jnp.dot(m1, we2[...], preferred_element_type=jnp.float32) + be2[...])
    i1 = _silu(
        jnp.dot(m, wi1[...], preferred_element_type=jnp.float32) + bi1[...])
    e = _sigm(jnp.dot(i1, wi2[...], preferred_element_type=jnp.float32)
              + bi2s[...])
    c1 = _silu(
        jnp.dot(m, wc1[...], preferred_element_type=jnp.float32) + bc1[...])
    phi = jnp.dot(c1, wc2[...], preferred_element_type=jnp.float32)
    xu16 = (e * phi) * ds + sel3[...]
    mx_ref[...] = jnp.concatenate(
        [e * m, xu16, jnp.zeros((_BE, _D - _H - 16), jnp.float32)], axis=1)


def _tc_edge(g, k17, sel3, we2, be2, wi1, bi1, wi2, bi2s, wc1, bc1, wc2):
    es = g.shape[0]
    grid = (es // _BE,)
    full = lambda r, c: pl.BlockSpec((r, c), lambda i: (0, 0))
    return pl.pallas_call(
        _edge_body,
        grid=grid,
        in_specs=[
            pl.BlockSpec((_BE, _D), lambda i: (i, 0)),
            full(17, _H), full(1, 16), full(_H, _H), full(1, _H),
            full(_H, _H // 2), full(1, _H // 2), full(_H // 2, 1), full(1, 1),
            full(_H, _H), full(1, _H), full(_H, 1),
        ],
        out_specs=pl.BlockSpec((_BE, _D), lambda i: (i, 0)),
        out_shape=jax.ShapeDtypeStruct((es, _D), jnp.float32),
    )(g, k17, sel3, we2, be2, wi1, bi1, wi2, bi2s, wc1, bc1, wc2)


def _sc_scatter(row, mx, zm, ebase):
    epw = mx.shape[0] // _NW
    nchunk = epw // _C
    mesh = plsc.VectorSubcoreMesh(core_axis_name="c", subcore_axis_name="s")
    scratch = (
        [pltpu.VMEM((_C,), jnp.int32) for _ in range(_NBUF)]
        + [pltpu.VMEM((_C, _TD), jnp.float32) for _ in range(_NBUF)]
        + [pltpu.SemaphoreType.DMA for _ in range(_NBUF)]
        + [pltpu.VMEM_SHARED((_N, _TD), jnp.float32)]
    )

    @functools.partial(
        pl.kernel,
        out_type=jax.ShapeDtypeStruct((_NC, _N, _D), jnp.float32),
        mesh=mesh,
        scratch_types=scratch,
        compiler_params=pltpu.CompilerParams(use_tc_tiling_on_sc=False),
    )
    def k(row_h, mx_h, zm_h, macc_h, *s):
        ibuf = s[0:_NBUF]
        mbuf = s[_NBUF:2 * _NBUF]
        sem = s[2 * _NBUF:3 * _NBUF]
        sh_m = s[3 * _NBUF]
        cid = lax.axis_index("c")
        sid = lax.axis_index("s")
        wid = cid * _NS + sid
        base = wid * epw

        @pl.when(sid == 0)
        def _init():
            pltpu.sync_copy(zm_h, sh_m)

        plsc.subcore_barrier()

        def group(g, carry):
            gets = []
            for b in range(_NBUF):
                j = g * _NBUF + b
                src = base + j * _C
                gets.append(pltpu.async_copy(
                    row_h.at[pl.ds(ebase + src, _C)], ibuf[b], sem[b]))
                gets.append(pltpu.async_copy(
                    mx_h.at[pl.ds(src, _C), pl.ds(0, _TD)], mbuf[b], sem[b]))
            for b in range(_NBUF):
                gets[2 * b].wait()
                gets[2 * b + 1].wait()
                pltpu.sync_copy(mbuf[b], sh_m.at[ibuf[b]], add=True)
            return carry

        lax.fori_loop(0, nchunk // _NBUF, group, 0)
        plsc.subcore_barrier()

        @pl.when(sid == 0)
        def _writeout():
            pltpu.sync_copy(sh_m, macc_h.at[cid, :, pl.ds(0, _TD)])

    return k(row, mx, zm)


def _final_body(hn_ref, *refs):
    (macc_refs, (x_ref, vout_ref, wn1b, bn1, wn2, bn2,
                 hout_ref, xout_ref)) = refs[:_P], refs[_P:]
    acc = macc_refs[0][0] + macc_refs[0][1]
    for r in macc_refs[1:]:
        acc = acc + r[0] + r[1]
    m_i = acc[:, :_H]
    t = jax.nn.silu(
        hn_ref[...]
        + jnp.dot(m_i, wn1b[...], preferred_element_type=jnp.float32)
        + bn1[...])
    hout_ref[...] = jnp.dot(t, wn2[...],
                            preferred_element_type=jnp.float32) + bn2[...]
    deg = acc[:, _H + 3:_H + 4]
    aggx = acc[:, _H:_H + 3]
    xb = x_ref[...]
    xout_ref[...] = jnp.where(
        deg > 0.0, xb + vout_ref[...] + aggx / jnp.float32(_N - 1), xb)


def _tc_final(hn, maccs, x, v_out, wn1b, bn1, wn2, bn2):
    grid = (_N // _BN,)
    full = lambda r, c: pl.BlockSpec((r, c), lambda i: (0, 0))
    return pl.pallas_call(
        _final_body,
        grid=grid,
        in_specs=[
            pl.BlockSpec((_BN, _H), lambda i: (i, 0)),
        ] + [
            pl.BlockSpec((_NC, _BN, _D), lambda i: (0, i, 0))
            for _ in range(_P)
        ] + [
            pl.BlockSpec((_BN, 3), lambda i: (i, 0)),
            pl.BlockSpec((_BN, 3), lambda i: (i, 0)),
            full(_H, _H), full(1, _H), full(_H, _D), full(1, _D),
        ],
        out_specs=[
            pl.BlockSpec((_BN, _D), lambda i: (i, 0)),
            pl.BlockSpec((_BN, 3), lambda i: (i, 0)),
        ],
        out_shape=[
            jax.ShapeDtypeStruct((_N, _D), jnp.float32),
            jax.ShapeDtypeStruct((_N, 3), jnp.float32),
        ],
    )(hn, *maccs, x, v_out, wn1b, bn1, wn2, bn2)


def kernel(h, x, edge_index, edge_attr, v_init, We1, be1, We2, be2, Wc1, bc1,
           Wc2, Wn1, bn1, Wn2, bn2, Wv1, bv1, Wv2, Wi1, bi1, Wi2, bi2):
    f32 = jnp.float32
    row = edge_index[0]
    col = edge_index[1]

    t1, t2, hn, v_out = _tc_pre(
        h, x, v_init,
        We1[:_D], be1.reshape(1, _H), We1[_D:2 * _D], Wn1[:_D],
        Wv1, bv1.reshape(1, _H), Wv2[:, 0].reshape(1, _H))

    k17 = jnp.concatenate(
        [jnp.tile(We1[2 * _D].reshape(1, _H), (3, 1)),
         jnp.zeros((13, _H), f32),
         We1[2 * _D + 1].reshape(1, _H)], axis=0)
    sel3 = jnp.zeros((1, 16), f32).at[0, 3].set(1.0)
    zm = jnp.zeros((_N, _TD), f32)

    maccs = []
    ebase = 0
    for s in range(_P):
        gx = _sc_gather(row, col, edge_attr.reshape(_E), t1, t2, ebase,
                        _SLICES[s])
        mx = _tc_edge(
            gx,
            k17, sel3,
            We2, be2.reshape(1, _H),
            Wi1, bi1.reshape(1, _H // 2), Wi2,
            bi2.reshape(1, 1),
            Wc1, bc1.reshape(1, _H), Wc2)
        maccs.append(_sc_scatter(row, mx, zm, ebase))
        ebase += _SLICES[s]

    h_out, x_out = _tc_final(
        hn, maccs, x, v_out,
        Wn1[_D:], bn1.reshape(1, _H), Wn2, bn2.reshape(1, _D))

    return (h_out, x_out, v_out)


# speedup vs baseline: 1.6310x; 1.0418x over previous
import functools

import jax
import jax.numpy as jnp
from jax import lax
from jax.experimental import pallas as pl
from jax.experimental.pallas import tpu as pltpu
from jax.experimental.pallas import tpu_sc as plsc

_N = 10000
_E = 320000
_D = 128
_H = 64
_TD = 80
_XD = 16

_NC, _NS = 2, 16
_NW = _NC * _NS
_SLICES = (96000, 128000, 96000)
_P = len(_SLICES)
_C = 40
_NBUF = 5
_BN = 1000
_BE = 4000


def _pre_body(h_ref, x_ref, vi_ref, we1a, be1, we1b, wn1a, wv1, bv1, wv2,
              t1_ref, t2_ref, hn_ref, vout_ref):
    hb = h_ref[...]
    xpad = jnp.concatenate(
        [x_ref[...], jnp.zeros((_BN, _XD - 3), jnp.float32)], axis=1)
    a1 = jnp.dot(hb, we1a[...], preferred_element_type=jnp.float32) + be1[...]
    a2 = jnp.dot(hb, we1b[...], preferred_element_type=jnp.float32)
    t1_ref[...] = jnp.concatenate([a1, xpad], axis=1)
    t2_ref[...] = jnp.concatenate([a2, xpad], axis=1)
    hn_ref[...] = jnp.dot(hb, wn1a[...], preferred_element_type=jnp.float32)
    vs = jax.nn.silu(
        jnp.dot(hb, wv1[...], preferred_element_type=jnp.float32) + bv1[...])
    vel = jnp.sum(vs * wv2[...], axis=1, keepdims=True)
    vout_ref[...] = vi_ref[...] * vel


def _tc_pre(h, x, v_init, we1a, be1, we1b, wn1a, wv1, bv1, wv2):
    grid = (_N // _BN,)
    full = lambda r, c: pl.BlockSpec((r, c), lambda i: (0, 0))
    return pl.pallas_call(
        _pre_body,
        grid=grid,
        in_specs=[
            pl.BlockSpec((_BN, _D), lambda i: (i, 0)),
            pl.BlockSpec((_BN, 3), lambda i: (i, 0)),
            pl.BlockSpec((_BN, 3), lambda i: (i, 0)),
            full(_D, _H), full(1, _H), full(_D, _H), full(_D, _H),
            full(_D, _H), full(1, _H), full(1, _H),
        ],
        out_specs=[
            pl.BlockSpec((_BN, _TD), lambda i: (i, 0)),
            pl.BlockSpec((_BN, _TD), lambda i: (i, 0)),
            pl.BlockSpec((_BN, _H), lambda i: (i, 0)),
            pl.BlockSpec((_BN, 3), lambda i: (i, 0)),
        ],
        out_shape=[
            jax.ShapeDtypeStruct((_N, _TD), jnp.float32),
            jax.ShapeDtypeStruct((_N, _TD), jnp.float32),
            jax.ShapeDtypeStruct((_N, _H), jnp.float32),
            jax.ShapeDtypeStruct((_N, 3), jnp.float32),
        ],
    )(h, x, v_init, we1a, be1, we1b, wn1a, wv1, bv1, wv2)


_GBUF = 5


def _sc_gather(row, col, ea, t1, t2, ebase, es):
    epw = es // _NW
    nchunk = epw // _C
    mesh = plsc.VectorSubcoreMesh(core_axis_name="c", subcore_axis_name="s")
    scratch = (
        [pltpu.VMEM((epw,), jnp.int32) for _ in range(2)]
        + [pltpu.VMEM((_C, _TD), jnp.float32) for _ in range(2 * _GBUF)]
        + [pltpu.VMEM((_C, _D), jnp.float32) for _ in range(_GBUF)]
        + [pltpu.VMEM((_C + 8,), jnp.float32) for _ in range(_GBUF)]
        + [pltpu.SemaphoreType.DMA for _ in range(2 * _GBUF)]
    )

    @functools.partial(
        pl.kernel,
        out_type=jax.ShapeDtypeStruct((es, _D), jnp.float32),
        mesh=mesh,
        scratch_types=scratch,
        compiler_params=pltpu.CompilerParams(use_tc_tiling_on_sc=False,
                                             needs_layout_passes=False),
    )
    def k(row_h, col_h, ea_h, t1_h, t2_h, mx_h, *s):
        idx_r, idx_c = s[0], s[1]
        bufs1 = s[2:2 + _GBUF]
        bufs2 = s[2 + _GBUF:2 + 2 * _GBUF]
        outb = s[2 + 2 * _GBUF:2 + 3 * _GBUF]
        eab = s[2 + 3 * _GBUF:2 + 4 * _GBUF]
        gsem = s[2 + 4 * _GBUF:2 + 5 * _GBUF]
        wsem = s[2 + 5 * _GBUF:2 + 6 * _GBUF]
        cid = lax.axis_index("c")
        sid = lax.axis_index("s")
        wid = cid * _NS + sid
        base = wid * epw
        pltpu.sync_copy(row_h.at[pl.ds(ebase + base, epw)], idx_r)
        pltpu.sync_copy(col_h.at[pl.ds(ebase + base, epw)], idx_c)
        iota16 = lax.iota(jnp.int32, 16)
        lane80 = jnp.full((16,), _TD, jnp.int32)
        m8 = iota16 < 8

        def issue_get(j, b):
            ofs = j * _C
            pltpu.async_copy(
                t1_h.at[idx_r.at[pl.ds(ofs, _C)]], bufs1[b], gsem[b])
            pltpu.async_copy(
                t2_h.at[idx_c.at[pl.ds(ofs, _C)]], bufs2[b], gsem[b])
            pltpu.async_copy(
                ea_h.at[pl.ds(ebase + base + ofs, _C)],
                eab[b].at[pl.ds(0, _C)], gsem[b])

        def wait_get(b):
            pltpu.make_async_copy(t1_h.at[pl.ds(0, _C)], bufs1[b],
                                  gsem[b]).wait()
            pltpu.make_async_copy(t2_h.at[pl.ds(0, _C)], bufs2[b],
                                  gsem[b]).wait()
            pltpu.make_async_copy(ea_h.at[pl.ds(0, _C)],
                                  eab[b].at[pl.ds(0, _C)], gsem[b]).wait()

        def wait_put(b):
            pltpu.make_async_copy(outb[b], mx_h.at[pl.ds(0, _C)],
                                  wsem[b]).wait()

        def compute(j, b):
            def edge(i, carry):
                d16 = (bufs1[b][i, pl.ds(_H, 16)]
                       - bufs2[b][i, pl.ds(_H, 16)])
                for q in range(4):
                    sl = pl.ds(q * 16, 16)
                    outb[b][i, sl] = bufs1[b][i, sl] + bufs2[b][i, sl]
                outb[b][i, pl.ds(_H, 16)] = d16
                return carry

            lax.fori_loop(0, _C, edge, 0)

            def eascat(k8, carry):
                ea16 = eab[b][pl.ds(k8 * 8, 16)]
                plsc.store_scatter(outb[b], [iota16 + k8 * 8, lane80],
                                   ea16, mask=m8)
                return carry

            lax.fori_loop(0, _C // 8, eascat, 0)
            pltpu.async_copy(outb[b], mx_h.at[pl.ds(base + j * _C, _C)],
                             wsem[b])

        for b in range(_GBUF):
            issue_get(b, b)

        def group(g, carry):
            for b in range(_GBUF):
                j = g * _GBUF + b
                wait_get(b)

                @pl.when(g > 0)
                def _drain():
                    wait_put(b)

                compute(j, b)

                @pl.when(j + _GBUF < nchunk)
                def _prefetch():
                    issue_get(j + _GBUF, b)

            return carry

        lax.fori_loop(0, nchunk // _GBUF, group, 0)
        for b in range(_GBUF):
            wait_put(b)

    return k(row, col, ea, t1, t2)


def _sigm(x):
    return 0.5 * jnp.tanh(0.5 * x) + 0.5


def _silu(x):
    return x * _sigm(x)


def _edge_body(g_ref, k17, sel3, we2, be2, wi1, bi1, wi2, bi2s,
               wc1, bc1, wc2, mx_ref):
    g = g_ref[...]
    ds = g[:, _H:_H + 16]
    dsea = jnp.concatenate([ds * ds, g[:, _TD:_TD + 1]], axis=1)
    epre = (g[:, :_H]
            + jnp.dot(dsea, k17[...], preferred_element_type=jnp.float32))
    m1 = _silu(epre)
    m = _silu(
        jnp.dot(m1, we2[...], preferred_element_type=jnp.float32) + be2[...])
    i1 = _silu(
        jnp.dot(m, wi1[...], preferred_element_type=jnp.float32) + bi1[...])
    e = _sigm(jnp.dot(i1, wi2[...], preferred_element_type=jnp.float32)
              + bi2s[...])
    c1 = _silu(
        jnp.dot(m, wc1[...], preferred_element_type=jnp.float32) + bc1[...])
    phi = jnp.dot(c1, wc2[...], preferred_element_type=jnp.float32)
    xu16 = (e * phi) * ds + sel3[...]
    mx_ref[...] = jnp.concatenate(
        [e * m, xu16, jnp.zeros((_BE, _D - _H - 16), jnp.float32)], axis=1)


def _tc_edge(g, k17, sel3, we2, be2, wi1, bi1, wi2, bi2s, wc1, bc1, wc2):
    es = g.shape[0]
    grid = (es // _BE,)
    full = lambda r, c: pl.BlockSpec((r, c), lambda i: (0, 0))
    return pl.pallas_call(
        _edge_body,
        grid=grid,
        in_specs=[
            pl.BlockSpec((_BE, _D), lambda i: (i, 0)),
            full(17, _H), full(1, 16), full(_H, _H), full(1, _H),
            full(_H, _H // 2), full(1, _H // 2), full(_H // 2, 1), full(1, 1),
            full(_H, _H), full(1, _H), full(_H, 1),
        ],
        out_specs=pl.BlockSpec((_BE, _D), lambda i: (i, 0)),
        out_shape=jax.ShapeDtypeStruct((es, _D), jnp.float32),
    )(g, k17, sel3, we2, be2, wi1, bi1, wi2, bi2s, wc1, bc1, wc2)


def _sc_scatter(row, mx, zm, ebase):
    epw = mx.shape[0] // _NW
    nchunk = epw // _C
    mesh = plsc.VectorSubcoreMesh(core_axis_name="c", subcore_axis_name="s")
    scratch = (
        [pltpu.VMEM((_C,), jnp.int32) for _ in range(_NBUF)]
        + [pltpu.VMEM((_C, _TD), jnp.float32) for _ in range(_NBUF)]
        + [pltpu.SemaphoreType.DMA for _ in range(_NBUF)]
        + [pltpu.VMEM_SHARED((_N, _TD), jnp.float32)]
    )

    @functools.partial(
        pl.kernel,
        out_type=jax.ShapeDtypeStruct((_NC, _N, _D), jnp.float32),
        mesh=mesh,
        scratch_types=scratch,
        compiler_params=pltpu.CompilerParams(use_tc_tiling_on_sc=False),
    )
    def k(row_h, mx_h, zm_h, macc_h, *s):
        ibuf = s[0:_NBUF]
        mbuf = s[_NBUF:2 * _NBUF]
        sem = s[2 * _NBUF:3 * _NBUF]
        sh_m = s[3 * _NBUF]
        cid = lax.axis_index("c")
        sid = lax.axis_index("s")
        wid = cid * _NS + sid
        base = wid * epw

        @pl.when(sid == 0)
        def _init():
            pltpu.sync_copy(zm_h, sh_m)

        plsc.subcore_barrier()

        def group(g, carry):
            gets = []
            for b in range(_NBUF):
                j = g * _NBUF + b
                src = base + j * _C
                gets.append(pltpu.async_copy(
                    row_h.at[pl.ds(ebase + src, _C)], ibuf[b], sem[b]))
                gets.append(pltpu.async_copy(
                    mx_h.at[pl.ds(src, _C), pl.ds(0, _TD)], mbuf[b], sem[b]))
            for b in range(_NBUF):
                gets[2 * b].wait()
                gets[2 * b + 1].wait()
                pltpu.sync_copy(mbuf[b], sh_m.at[ibuf[b]], add=True)
            return carry

        lax.fori_loop(0, nchunk // _NBUF, group, 0)
        plsc.subcore_barrier()

        @pl.when(sid == 0)
        def _writeout():
            pltpu.sync_copy(sh_m, macc_h.at[cid, :, pl.ds(0, _TD)])

    return k(row, mx, zm)


def _final_body(hn_ref, *refs):
    (macc_refs, (x_ref, vout_ref, wn1b, bn1, wn2, bn2,
                 hout_ref, xout_ref)) = refs[:_P], refs[_P:]
    acc = macc_refs[0][0] + macc_refs[0][1]
    for r in macc_refs[1:]:
        acc = acc + r[0] + r[1]
    m_i = acc[:, :_H]
    t = jax.nn.silu(
        hn_ref[...]
        + jnp.dot(m_i, wn1b[...], preferred_element_type=jnp.float32)
        + bn1[...])
    hout_ref[...] = jnp.dot(t, wn2[...],
                            preferred_element_type=jnp.float32) + bn2[...]
    deg = acc[:, _H + 3:_H + 4]
    aggx = acc[:, _H:_H + 3]
    xb = x_ref[...]
    xout_ref[...] = jnp.where(
        deg > 0.0, xb + vout_ref[...] + aggx / jnp.float32(_N - 1), xb)


def _tc_final(hn, maccs, x, v_out, wn1b, bn1, wn2, bn2):
    grid = (_N // _BN,)
    full = lambda r, c: pl.BlockSpec((r, c), lambda i: (0, 0))
    return pl.pallas_call(
        _final_body,
        grid=grid,
        in_specs=[
            pl.BlockSpec((_BN, _H), lambda i: (i, 0)),
        ] + [
            pl.BlockSpec((_NC, _BN, _D), lambda i: (0, i, 0))
            for _ in range(_P)
        ] + [
            pl.BlockSpec((_BN, 3), lambda i: (i, 0)),
            pl.BlockSpec((_BN, 3), lambda i: (i, 0)),
            full(_H, _H), full(1, _H), full(_H, _D), full(1, _D),
        ],
        out_specs=[
            pl.BlockSpec((_BN, _D), lambda i: (i, 0)),
            pl.BlockSpec((_BN, 3), lambda i: (i, 0)),
        ],
        out_shape=[
            jax.ShapeDtypeStruct((_N, _D), jnp.float32),
            jax.ShapeDtypeStruct((_N, 3), jnp.float32),
        ],
    )(hn, *maccs, x, v_out, wn1b, bn1, wn2, bn2)


def kernel(h, x, edge_index, edge_attr, v_init, We1, be1, We2, be2, Wc1, bc1,
           Wc2, Wn1, bn1, Wn2, bn2, Wv1, bv1, Wv2, Wi1, bi1, Wi2, bi2):
    f32 = jnp.float32
    row = edge_index[0]
    col = edge_index[1]

    t1, t2, hn, v_out = _tc_pre(
        h, x, v_init,
        We1[:_D], be1.reshape(1, _H), We1[_D:2 * _D], Wn1[:_D],
        Wv1, bv1.reshape(1, _H), Wv2[:, 0].reshape(1, _H))

    k17 = jnp.concatenate(
        [jnp.tile(We1[2 * _D].reshape(1, _H), (3, 1)),
         jnp.zeros((13, _H), f32),
         We1[2 * _D + 1].reshape(1, _H)], axis=0)
    sel3 = jnp.zeros((1, 16), f32).at[0, 3].set(1.0)
    zm = jnp.zeros((_N, _TD), f32)

    maccs = []
    ebase = 0
    for s in range(_P):
        gx = _sc_gather(row, col, edge_attr.reshape(_E), t1, t2, ebase,
                        _SLICES[s])
        mx = _tc_edge(
            gx,
            k17, sel3,
            We2, be2.reshape(1, _H),
            Wi1, bi1.reshape(1, _H // 2), Wi2,
            bi2.reshape(1, 1),
            Wc1, bc1.reshape(1, _H), Wc2)
        maccs.append(_sc_scatter(row, mx, zm, ebase))
        ebase += _SLICES[s]

    h_out, x_out = _tc_final(
        hn, maccs, x, v_out,
        Wn1[_D:], bn1.reshape(1, _H), Wn2, bn2.reshape(1, _D))

    return (h_out, x_out, v_out)
